# Initial kernel scaffold; baseline (speedup 1.0000x reference)
import jax
import jax.numpy as jnp
from jax.experimental import pallas as pl


def kernel(x, edge_index, W1, b1, W2, b2):
    raise NotImplementedError("write your pallas kernel here")


# speedup vs baseline: 5.9940x; 5.9940x over previous
import functools

import jax
import jax.numpy as jnp
from jax import lax
from jax.experimental import pallas as pl
from jax.experimental.pallas import tpu as pltpu
from jax.experimental.pallas import tpu_sc as plsc

N = 10000
NP = 10240
E = 160000
EP = 163840
STRIPE = NP // 16

_MESH = plsc.VectorSubcoreMesh(
    core_axis_name="c", subcore_axis_name="s", num_cores=2, num_subcores=16)


@functools.partial(
    pl.kernel,
    out_type=jax.ShapeDtypeStruct((2, NP), jnp.float32),
    mesh=_MESH,
    scratch_types=[
        pltpu.VMEM((40, 128), jnp.int32),
        pltpu.VMEM((128,), jnp.float32),
        pltpu.VMEM_SHARED((NP,), jnp.float32),
    ],
)
def _deg(col_hbm, zeros1_hbm, out_hbm, col_v, ones_v, deg_sh):
    c = lax.axis_index("c")
    s = lax.axis_index("s")
    pltpu.sync_copy(col_hbm.at[c, s], col_v)

    @pl.loop(0, 8)
    def _fill(i):
        ones_v[pl.ds(i * 16, 16)] = jnp.full((16,), 1.0, jnp.float32)

    pltpu.sync_copy(zeros1_hbm.at[pl.ds(s * STRIPE, STRIPE)],
                    deg_sh.at[pl.ds(s * STRIPE, STRIPE)])
    plsc.subcore_barrier()

    @pl.loop(0, 40)
    def _scat(g):
        pltpu.sync_copy(ones_v, deg_sh.at[col_v.at[g]], add=True)

    plsc.subcore_barrier()
    pltpu.sync_copy(deg_sh.at[pl.ds(s * STRIPE, STRIPE)],
                    out_hbm.at[c, pl.ds(s * STRIPE, STRIPE)])


def _make_spmm(n_chunks):
    @functools.partial(
        pl.kernel,
        out_type=jax.ShapeDtypeStruct((2, NP, 128), jnp.float32),
        mesh=_MESH,
        scratch_types=[
            pltpu.VMEM((n_chunks, 128), jnp.int32),
            pltpu.VMEM((n_chunks, 128), jnp.int32),
            pltpu.VMEM((128, 128), jnp.float32),
            pltpu.VMEM((128, 128), jnp.float32),
            pltpu.VMEM_SHARED((NP, 128), jnp.float32),
            pltpu.SemaphoreType.DMA,
            pltpu.SemaphoreType.DMA,
        ],
    )
    def spmm(row_hbm, col_hbm, table_hbm, zeros2_hbm, out_hbm,
             row_v, col_v, buf_a, buf_b, acc_sh, sem_a, sem_b):
        c = lax.axis_index("c")
        s = lax.axis_index("s")
        pltpu.sync_copy(row_hbm.at[c, s], row_v)
        pltpu.sync_copy(col_hbm.at[c, s], col_v)
        pltpu.sync_copy(zeros2_hbm, acc_sh.at[pl.ds(s * STRIPE, STRIPE)])
        plsc.subcore_barrier()

        @pl.loop(0, n_chunks)
        def _chunks(g):
            pltpu.async_copy(table_hbm.at[row_v.at[g]], buf_a, sem_a).wait()
            pltpu.sync_copy(buf_a, acc_sh.at[col_v.at[g]], add=True)

        plsc.subcore_barrier()
        pltpu.sync_copy(acc_sh.at[pl.ds(s * STRIPE, STRIPE)],
                        out_hbm.at[c, pl.ds(s * STRIPE, STRIPE)])

    return spmm


_spmm80 = _make_spmm(80)
_spmm40 = _make_spmm(40)


def _mm_body(x_ref, w_ref, o_ref):
    o_ref[...] = jnp.dot(x_ref[...], w_ref[...],
                         preferred_element_type=jnp.float32)


def _matmul1(xp, w1):
    return pl.pallas_call(
        _mm_body,
        grid=(NP // 128,),
        in_specs=[pl.BlockSpec((128, 256), lambda i: (i, 0)),
                  pl.BlockSpec((256, 256), lambda i: (0, 0))],
        out_specs=pl.BlockSpec((128, 256), lambda i: (i, 0)),
        out_shape=jax.ShapeDtypeStruct((NP, 256), jnp.float32),
    )(xp, w1)


def _prescale_body(dega_ref, degb_ref, h_ref, ht0_ref, ht1_ref, dinv_ref):
    deg = dega_ref[...] + degb_ref[...] + 1.0
    d = jnp.broadcast_to(lax.rsqrt(deg), (128, 128))
    h = h_ref[...]
    ht0_ref[...] = h[:, :128] * d
    ht1_ref[...] = h[:, 128:] * d
    dinv_ref[...] = d


def _prescale(dega, degb, h1p):
    return pl.pallas_call(
        _prescale_body,
        grid=(NP // 128,),
        in_specs=[pl.BlockSpec((128, 1), lambda i: (i, 0)),
                  pl.BlockSpec((128, 1), lambda i: (i, 0)),
                  pl.BlockSpec((128, 256), lambda i: (i, 0))],
        out_specs=[pl.BlockSpec((128, 128), lambda i: (i, 0))] * 3,
        out_shape=[jax.ShapeDtypeStruct((NP, 128), jnp.float32)] * 3,
    )(dega, degb, h1p)


def _mid_body(a0_ref, a1_ref, t0_ref, t1_ref, dinv_ref, b0_ref, b1_ref,
              w_ref, o_ref):
    d = dinv_ref[...]
    r0 = jnp.maximum((a0_ref[...] + t0_ref[...]) * d + b0_ref[...], 0.0)
    r1 = jnp.maximum((a1_ref[...] + t1_ref[...]) * d + b1_ref[...], 0.0)
    w = w_ref[...]
    h2 = (jnp.dot(r0, w[:128, :], preferred_element_type=jnp.float32)
          + jnp.dot(r1, w[128:, :], preferred_element_type=jnp.float32))
    o_ref[...] = h2 * d


def _mid(a0, a1, t0, t1, dinv, b0r, b1r, w2):
    spec128 = pl.BlockSpec((128, 128), lambda i: (i, 0))
    return pl.pallas_call(
        _mid_body,
        grid=(NP // 128,),
        in_specs=[spec128, spec128, spec128, spec128, spec128,
                  pl.BlockSpec((1, 128), lambda i: (0, 0)),
                  pl.BlockSpec((1, 128), lambda i: (0, 0)),
                  pl.BlockSpec((256, 128), lambda i: (0, 0))],
        out_specs=spec128,
        out_shape=jax.ShapeDtypeStruct((NP, 128), jnp.float32),
    )(a0, a1, t0, t1, dinv, b0r, b1r, w2)


def _final_body(a0_ref, a1_ref, t_ref, dinv_ref, b_ref, o_ref):
    o_ref[...] = ((a0_ref[...] + a1_ref[...] + t_ref[...]) * dinv_ref[...]
                  + b_ref[...])


def _final(a0, a1, t, dinv, b2r):
    spec128 = pl.BlockSpec((128, 128), lambda i: (i, 0))
    return pl.pallas_call(
        _final_body,
        grid=(NP // 128,),
        in_specs=[spec128, spec128, spec128, spec128,
                  pl.BlockSpec((1, 128), lambda i: (0, 0))],
        out_specs=spec128,
        out_shape=jax.ShapeDtypeStruct((NP, 128), jnp.float32),
    )(a0, a1, t, dinv, b2r)


def kernel(x, edge_index, W1, b1, W2, b2):
    ei = edge_index.astype(jnp.int32)
    pad = jnp.full((EP - E,), N, jnp.int32)
    rowp = jnp.concatenate([ei[0], pad])
    colp = jnp.concatenate([ei[1], pad])

    row_l1h = rowp.reshape(16, 80, 128)
    row_l1 = jnp.stack([row_l1h, row_l1h + NP])
    col_l1 = jnp.broadcast_to(colp.reshape(1, 16, 80, 128),
                              (2, 16, 80, 128))
    row_l2 = rowp.reshape(2, 16, 40, 128)
    col_l2 = colp.reshape(2, 16, 40, 128)

    zeros1 = jnp.zeros((NP,), jnp.float32)
    zeros2 = jnp.zeros((STRIPE, 128), jnp.float32)
    xp = jnp.pad(x, ((0, NP - N), (0, 0)))

    deg = _deg(col_l2, zeros1)
    h1p = _matmul1(xp, W1)
    ht0, ht1, dinv = _prescale(deg[0].reshape(NP, 1),
                               deg[1].reshape(NP, 1), h1p)
    table1 = jnp.concatenate([ht0, ht1], axis=0)
    acc1 = _spmm80(row_l1, col_l1, table1, zeros2)
    ht2 = _mid(acc1[0], acc1[1], ht0, ht1, dinv,
               b1[:128].reshape(1, 128), b1[128:].reshape(1, 128), W2)
    acc2 = _spmm40(row_l2, col_l2, ht2, zeros2)
    outp = _final(acc2[0], acc2[1], ht2, dinv, b2.reshape(1, 128))
    return outp[:N]
